# baseline (speedup 1.0000x reference)
import jax
import jax.numpy as jnp
import numpy as np
from jax.experimental import pallas as pl

B, H, W = 1, 96, 160
D = 48
CF4 = 96
CP = 12
CGW = 8
CV = 8
HID = 128
RAD = 4
K = 2 * RAD + 1
LEVELS = 2
ITERS = 4
CM = 32
CGEO = LEVELS * (CV * K + K)


def _conv2d(x, w, b=None, stride=1):
    y = jax.lax.conv_general_dilated(x, w, (stride, stride), 'SAME',
                                     dimension_numbers=('NCHW', 'OIHW', 'NCHW'))
    return y if b is None else y + b[None, :, None, None]


def _conv3d(x, w, b=None):
    y = jax.lax.conv_general_dilated(x, w, (1, 1, 1), 'SAME',
                                     dimension_numbers=('NCDHW', 'OIDHW', 'NCDHW'))
    return y if b is None else y + b[None, :, None, None, None]


def _linsample(vol, pos):
    L = vol.shape[-1]
    x0 = jnp.floor(pos)
    f = (pos - x0).astype(vol.dtype)
    i0 = jnp.clip(x0.astype(jnp.int32), 0, L - 1)
    i1 = jnp.clip(i0 + 1, 0, L - 1)
    return jnp.take_along_axis(vol, i0, -1) * (1 - f) + jnp.take_along_axis(vol, i1, -1) * f


def _upsample_kernel(unf_ref, spx_ref, o_ref):
    o_ref[...] = jnp.sum(unf_ref[...] * spx_ref[...], axis=0, keepdims=True)


def _taps(disp, scale, L, iota):
    dl = disp * scale
    x0 = jnp.floor(dl)
    f = dl - x0
    w0 = []
    w1 = []
    for k in range(K):
        i0 = jnp.clip(x0 + (k - RAD), 0.0, L - 1.0)
        i1 = jnp.minimum(i0 + 1.0, L - 1.0)
        w0.append((iota == i0).astype(jnp.float32))
        w1.append((iota == i1).astype(jnp.float32))
    return w0, w1, f


def _lookup_kernel(gv_ref, co_ref, disp_ref, mw_ref, mb_ref, o_ref):
    disp = disp_ref[...]
    gv = gv_ref[...]
    co = co_ref[...]
    pieces = []
    gv_off = [0, 48]
    co_off = [0, 160]
    for l in range(LEVELS):
        scale = 1.0 / (2 ** l)
        Lg = D >> l
        g = gv[:, :, gv_off[l]:gv_off[l] + Lg]
        iota3 = jax.lax.broadcasted_iota(jnp.int32, (1, 1, Lg), 2).astype(jnp.float32)
        w0, w1, f = _taps(disp[:, :, None], scale, Lg, iota3)
        f3 = f
        for k in range(K):
            wv = w0[k] * (1.0 - f3) + w1[k] * f3
            pieces.append(jnp.sum(g * wv, axis=-1))
        Lc = W >> l
        c = co[:, co_off[l]:co_off[l] + Lc]
        iota2 = jax.lax.broadcasted_iota(jnp.int32, (1, Lc), 1).astype(jnp.float32)
        rows = gv.shape[0]
        ri = jax.lax.broadcasted_iota(jnp.int32, (rows, 1), 0).astype(jnp.float32)
        wcoord = ri - W * jnp.floor(ri / W)
        cd = (wcoord - disp)
        w0c, w1c, fc = _taps(cd, scale, Lc, iota2)
        acc = []
        for k in range(K):
            wv = w0c[k] * (1.0 - fc) + w1c[k] * fc
            acc.append(jnp.sum(c * wv, axis=-1, keepdims=True))
        pieces.append(jnp.concatenate(acc, axis=-1))
    geo = jnp.concatenate(pieces, axis=-1)
    o_ref[...] = jax.nn.relu(jnp.dot(geo, mw_ref[...],
                                     preferred_element_type=jnp.float32) + mb_ref[...])


def kernel(features_left_04, features_left_08, features_left_16, features_left_32,
           features_right_04, stem_2x, gwc_volume, params):
    f4 = features_left_04
    fr4 = features_right_04
    p = params
    left = _conv2d(f4, p['proj_w'], p['proj_b'])
    right = _conv2d(fr4, p['proj_w'], p['proj_b'])
    dax = jnp.arange(D); wax = jnp.arange(W)
    mask = (wax[None, :] >= dax[:, None]).astype(left.dtype)
    ridx = jnp.clip(wax[None, :] - dax[:, None], 0, W - 1)
    lvol = left[:, :, None, :, :] * mask[None, None, :, None, :]
    rvol = jnp.moveaxis(right[:, :, :, ridx], 3, 2) * mask[None, None, :, None, :]
    comb = jnp.concatenate([gwc_volume, lvol, rvol], 1)
    comb = jax.nn.relu(_conv3d(comb, p['stem_w'], p['stem_b']))
    att2d = jax.nn.sigmoid(_conv2d(f4, p['att_w'], p['att_b']))
    comb = comb * att2d[:, :, None]
    comb = jax.nn.relu(_conv3d(comb, p['agg_w'], p['agg_b']))
    logits = _conv3d(comb, p['cls_w'], p['cls_b'])[:, 0]
    prob = jax.nn.softmax(logits, axis=1)
    init_disp = jnp.sum(prob * jnp.arange(D, dtype=prob.dtype)[None, :, None, None], 1)
    net = jnp.tanh(_conv2d(f4, p['cnet_net_w'], p['cnet_net_b']))
    inp = jax.nn.relu(_conv2d(f4, p['cnet_inp_w'], p['cnet_inp_b']))
    inp = inp * jax.nn.sigmoid(_conv2d(inp.mean((2, 3), keepdims=True), p['cam_w'], p['cam_b']))
    satt = jax.nn.sigmoid(_conv2d(inp, p['sam_w'], p['sam_b']))
    gv0 = jnp.transpose(comb, (0, 3, 4, 1, 2))
    gv1 = gv0.reshape(B, H, W, CV, D // 2, 2).mean(-1)
    corr0 = jnp.einsum('bchw,bchx->bhwx', f4, fr4) / np.sqrt(CF4)
    corr1 = corr0.reshape(B, H, W, W // 2, 2).mean(-1)
    gvcat = jnp.concatenate([gv0.reshape(H * W, CV, D),
                             gv1.reshape(H * W, CV, D // 2)], -1)
    cocat = jnp.concatenate([corr0.reshape(H * W, W),
                             corr1.reshape(H * W, W // 2)], -1)
    perm = []
    for base in (0, CV * K + K):
        for k in range(K):
            for c in range(CV):
                perm.append(base + c * K + k)
        for k in range(K):
            perm.append(base + CV * K + k)
    mw = jnp.transpose(p['meg_w'][:, :, 0, 0][:, jnp.array(perm)], (1, 0))
    mb = p['meg_b'][None, :]
    R = (H * W) // 32
    disp = init_disp
    for _ in range(ITERS):
        disp = jax.lax.stop_gradient(disp)
        mo_g = pl.pallas_call(
            _lookup_kernel,
            out_shape=jax.ShapeDtypeStruct((H * W, 96), jnp.float32),
            grid=(32,),
            in_specs=[
                pl.BlockSpec((R, CV, D + D // 2), lambda i: (i, 0, 0)),
                pl.BlockSpec((R, W + W // 2), lambda i: (i, 0)),
                pl.BlockSpec((R, 1), lambda i: (i, 0)),
                pl.BlockSpec((CGEO, 96), lambda i: (0, 0)),
                pl.BlockSpec((1, 96), lambda i: (0, 0)),
            ],
            out_specs=pl.BlockSpec((R, 96), lambda i: (i, 0)),
        )(gvcat, cocat, disp.reshape(H * W, 1), mw, mb)
        mg = jnp.transpose(mo_g.reshape(1, H, W, 96), (0, 3, 1, 2))
        md = jax.nn.relu(_conv2d(disp[:, None], p['med_w'], p['med_b']))
        mo = jax.nn.relu(_conv2d(jnp.concatenate([mg, md], 1), p['meo_w'], p['meo_b']))
        motion = jnp.concatenate([mo, disp[:, None]], 1)
        x = jnp.concatenate([inp, motion * satt], 1)
        hx = jnp.concatenate([net, x], 1)
        z = jax.nn.sigmoid(_conv2d(hx, p['gru_z_w'], p['gru_z_b']))
        r = jax.nn.sigmoid(_conv2d(hx, p['gru_r_w'], p['gru_r_b']))
        q = jnp.tanh(_conv2d(jnp.concatenate([r * net, x], 1), p['gru_q_w'], p['gru_q_b']))
        net = (1 - z) * net + z * q
        disp = disp + _conv2d(net, p['head_w'], p['head_b'])[:, 0]
    mask_feat = jax.nn.relu(_conv2d(net, p['mask_w'], p['mask_b']))
    mf2 = jnp.repeat(jnp.repeat(mask_feat, 2, 2), 2, 3)
    xspx = jax.nn.relu(_conv2d(jnp.concatenate([mf2, stem_2x], 1), p['spx2_w'], p['spx2_b']))
    spx = jax.lax.conv_transpose(xspx, p['spxg_w'], (2, 2), 'SAME',
                                 dimension_numbers=('NCHW', 'OIHW', 'NCHW'))
    spx = jax.nn.softmax(spx + p['spxg_b'][None, :, None, None], axis=1)
    dlow = (disp * 4.0)[:, None]
    dpad = jnp.pad(dlow, ((0, 0), (0, 0), (1, 1), (1, 1)))
    unf = jnp.concatenate([dpad[:, :, i:i + H, j:j + W]
                           for i in range(3) for j in range(3)], 1)
    unf = jnp.repeat(jnp.repeat(unf, 4, 2), 4, 3)
    out = pl.pallas_call(
        _upsample_kernel,
        out_shape=jax.ShapeDtypeStruct((1, 4 * H, 4 * W), jnp.float32),
        grid=(4,),
        in_specs=[
            pl.BlockSpec((9, H, 4 * W), lambda i: (0, i, 0)),
            pl.BlockSpec((9, H, 4 * W), lambda i: (0, i, 0)),
        ],
        out_specs=pl.BlockSpec((1, H, 4 * W), lambda i: (0, i, 0)),
    )(unf[0], spx[0])
    return out[None]


# speedup vs baseline: 1.0141x; 1.0141x over previous
import jax
import jax.numpy as jnp
import numpy as np
from jax.experimental import pallas as pl

B, H, W = 1, 96, 160
D = 48
CF4 = 96
CP = 12
CGW = 8
CV = 8
HID = 128
RAD = 4
K = 2 * RAD + 1
LEVELS = 2
ITERS = 4
CM = 32
CGEO = LEVELS * (CV * K + K)
HB = 8


def _conv2d(x, w, b=None, stride=1):
    y = jax.lax.conv_general_dilated(x, w, (stride, stride), 'SAME',
                                     dimension_numbers=('NCHW', 'OIHW', 'NCHW'))
    return y if b is None else y + b[None, :, None, None]


def _conv3d(x, w, b=None):
    y = jax.lax.conv_general_dilated(x, w, (1, 1, 1), 'SAME',
                                     dimension_numbers=('NCDHW', 'OIDHW', 'NCDHW'))
    return y if b is None else y + b[None, :, None, None, None]


def _tap_weights(dl, L, iota, k):
    x0 = jnp.floor(dl)
    f = dl - x0
    i0 = jnp.clip(x0 + (k - RAD), 0.0, L - 1.0)
    i1 = jnp.minimum(i0 + 1.0, L - 1.0)
    return (iota == i0).astype(jnp.float32) * (1.0 - f) + \
           (iota == i1).astype(jnp.float32) * f


def _lookup_kernel(gv_ref, co_ref, disp_ref, mw_ref, mb_ref, o_ref):
    g0 = gv_ref[...]
    g0p = g0.reshape(CV, D // 2, 2, HB, W)
    g1 = 0.5 * (g0p[:, :, 0] + g0p[:, :, 1])
    c0 = co_ref[...]
    iota_u = jax.lax.broadcasted_iota(jnp.int32, (W, W // 2), 0).astype(jnp.float32)
    iota_v = jax.lax.broadcasted_iota(jnp.int32, (W, W // 2), 1).astype(jnp.float32)
    avg = (((iota_u == 2.0 * iota_v) | (iota_u == 2.0 * iota_v + 1.0))
           .astype(jnp.float32) * 0.5)
    c1 = jnp.stack([jnp.dot(c0[h], avg, preferred_element_type=jnp.float32)
                    for h in range(HB)], axis=0)
    disp = disp_ref[...]
    d4 = disp[None, None, :, :]
    gv_pieces = []
    co_pieces = []
    for l in range(LEVELS):
        scale = 1.0 / (2 ** l)
        Lg = D >> l
        g = g0 if l == 0 else g1
        iota_d = jax.lax.broadcasted_iota(
            jnp.int32, (1, Lg, 1, 1), 1).astype(jnp.float32)
        lvl_gv = []
        for k in range(K):
            wv = _tap_weights(d4 * scale, Lg, iota_d, k)
            lvl_gv.append(jnp.sum(g * wv, axis=1))
        gv_pieces.append(lvl_gv)
        Lc = W >> l
        c = c0 if l == 0 else c1
        iota_w = jax.lax.broadcasted_iota(
            jnp.int32, (1, 1, Lc), 2).astype(jnp.float32)
        wcoord = jax.lax.broadcasted_iota(
            jnp.int32, (1, W, 1), 1).astype(jnp.float32)
        cd = (wcoord - disp[:, :, None]) * scale
        lvl_co = []
        for k in range(K):
            wv = _tap_weights(cd, Lc, iota_w, k)
            lvl_co.append(jnp.sum(c * wv, axis=-1))
        co_pieces.append(lvl_co)
    mwt = mw_ref[...]
    mb = mb_ref[...]
    for h in range(HB):
        rows = []
        for l in range(LEVELS):
            for k in range(K):
                rows.append(gv_pieces[l][k][:, h, :])
            for k in range(K):
                rows.append(co_pieces[l][k][h:h + 1, :])
        geo_h = jnp.concatenate(rows, axis=0)
        o_ref[:, h, :] = jax.nn.relu(
            jnp.dot(mwt, geo_h, preferred_element_type=jnp.float32) + mb)


def kernel(features_left_04, features_left_08, features_left_16, features_left_32,
           features_right_04, stem_2x, gwc_volume, params):
    f4 = features_left_04
    fr4 = features_right_04
    p = params
    left = _conv2d(f4, p['proj_w'], p['proj_b'])
    right = _conv2d(fr4, p['proj_w'], p['proj_b'])
    dax = jnp.arange(D); wax = jnp.arange(W)
    mask = (wax[None, :] >= dax[:, None]).astype(left.dtype)
    ridx = jnp.clip(wax[None, :] - dax[:, None], 0, W - 1)
    lvol = left[:, :, None, :, :] * mask[None, None, :, None, :]
    rvol = jnp.moveaxis(right[:, :, :, ridx], 3, 2) * mask[None, None, :, None, :]
    comb = jnp.concatenate([gwc_volume, lvol, rvol], 1)
    comb = jax.nn.relu(_conv3d(comb, p['stem_w'], p['stem_b']))
    att2d = jax.nn.sigmoid(_conv2d(f4, p['att_w'], p['att_b']))
    comb = comb * att2d[:, :, None]
    comb = jax.nn.relu(_conv3d(comb, p['agg_w'], p['agg_b']))
    logits = _conv3d(comb, p['cls_w'], p['cls_b'])[:, 0]
    prob = jax.nn.softmax(logits, axis=1)
    init_disp = jnp.sum(prob * jnp.arange(D, dtype=prob.dtype)[None, :, None, None], 1)
    net = jnp.tanh(_conv2d(f4, p['cnet_net_w'], p['cnet_net_b']))
    inp = jax.nn.relu(_conv2d(f4, p['cnet_inp_w'], p['cnet_inp_b']))
    inp = inp * jax.nn.sigmoid(_conv2d(inp.mean((2, 3), keepdims=True), p['cam_w'], p['cam_b']))
    satt = jax.nn.sigmoid(_conv2d(inp, p['sam_w'], p['sam_b']))
    corr0 = jnp.einsum('bchw,bchx->bhwx', f4, fr4) / np.sqrt(CF4)
    perm = []
    for base in (0, CV * K + K):
        for k in range(K):
            for c in range(CV):
                perm.append(base + c * K + k)
        for k in range(K):
            perm.append(base + CV * K + k)
    mw = p['meg_w'][:, np.array(perm), 0, 0]
    mb = p['meg_b'][:, None]
    comb8 = comb[0]
    co3 = corr0[0]
    disp = init_disp
    for _ in range(ITERS):
        disp = jax.lax.stop_gradient(disp)
        mg = pl.pallas_call(
            _lookup_kernel,
            out_shape=jax.ShapeDtypeStruct((96, H, W), jnp.float32),
            grid=(H // HB,),
            in_specs=[
                pl.BlockSpec((CV, D, HB, W), lambda i: (0, 0, i, 0)),
                pl.BlockSpec((HB, W, W), lambda i: (i, 0, 0)),
                pl.BlockSpec((HB, W), lambda i: (i, 0)),
                pl.BlockSpec((96, CGEO), lambda i: (0, 0)),
                pl.BlockSpec((96, 1), lambda i: (0, 0)),
            ],
            out_specs=pl.BlockSpec((96, HB, W), lambda i: (0, i, 0)),
        )(comb8, co3, disp[0], mw, mb)[None]
        md = jax.nn.relu(_conv2d(disp[:, None], p['med_w'], p['med_b']))
        mo = jax.nn.relu(_conv2d(jnp.concatenate([mg, md], 1), p['meo_w'], p['meo_b']))
        motion = jnp.concatenate([mo, disp[:, None]], 1)
        x = jnp.concatenate([inp, motion * satt], 1)
        hx = jnp.concatenate([net, x], 1)
        z = jax.nn.sigmoid(_conv2d(hx, p['gru_z_w'], p['gru_z_b']))
        r = jax.nn.sigmoid(_conv2d(hx, p['gru_r_w'], p['gru_r_b']))
        q = jnp.tanh(_conv2d(jnp.concatenate([r * net, x], 1), p['gru_q_w'], p['gru_q_b']))
        net = (1 - z) * net + z * q
        disp = disp + _conv2d(net, p['head_w'], p['head_b'])[:, 0]
    mask_feat = jax.nn.relu(_conv2d(net, p['mask_w'], p['mask_b']))
    mf2 = jnp.repeat(jnp.repeat(mask_feat, 2, 2), 2, 3)
    xspx = jax.nn.relu(_conv2d(jnp.concatenate([mf2, stem_2x], 1), p['spx2_w'], p['spx2_b']))
    spx = jax.lax.conv_transpose(xspx, p['spxg_w'], (2, 2), 'SAME',
                                 dimension_numbers=('NCHW', 'OIHW', 'NCHW'))
    spx = jax.nn.softmax(spx + p['spxg_b'][None, :, None, None], axis=1)
    dlow = (disp * 4.0)[:, None]
    dpad = jnp.pad(dlow, ((0, 0), (0, 0), (1, 1), (1, 1)))
    unf = jnp.concatenate([dpad[:, :, i:i + H, j:j + W]
                           for i in range(3) for j in range(3)], 1)
    unf = jnp.repeat(jnp.repeat(unf, 4, 2), 4, 3)
    return jnp.sum(unf * spx, 1, keepdims=True)
